# baseline (speedup 1.0000x reference)
import jax
import jax.numpy as jnp
from jax.experimental import pallas as pl
from jax.experimental.pallas import tpu as pltpu

_B, _S, _D, _F, _E = 4, 2048, 1024, 4096, 8
_BF = 512


def _gate_kernel(h_ref, wg_ref, logits_ref, choice_ref):
    hbar = jnp.mean(h_ref[...], axis=1)
    logits = jnp.dot(hbar, wg_ref[...], preferred_element_type=jnp.float32)
    logits_ref[...] = logits
    row_max = jnp.max(logits, axis=-1, keepdims=True)
    idx = jax.lax.broadcasted_iota(jnp.int32, logits.shape, 1)
    masked = jnp.where(logits == row_max, idx, _E)
    choice_ref[...] = jnp.min(masked, axis=-1, keepdims=True)


def _moe_kernel(choice_ref, h_ref, w1_ref, w2_ref, out_ref, xbf_ref):
    fj = pl.program_id(1)

    @pl.when(fj == 0)
    def _init():
        out_ref[...] = jnp.zeros_like(out_ref)
        xbf_ref[...] = h_ref[0].astype(jnp.bfloat16)

    hmid = jax.nn.gelu(
        jnp.dot(
            xbf_ref[...],
            w1_ref[0].astype(jnp.bfloat16),
            preferred_element_type=jnp.float32,
        )
    )
    out_ref[0, :, :] += jnp.dot(
        hmid.astype(jnp.bfloat16),
        w2_ref[0].astype(jnp.bfloat16),
        preferred_element_type=jnp.float32,
    )


def kernel(hidden_states, Wg, W1, W2):
    logits, choice = pl.pallas_call(
        _gate_kernel,
        out_shape=(
            jax.ShapeDtypeStruct((_B, _E), jnp.float32),
            jax.ShapeDtypeStruct((_B, 1), jnp.int32),
        ),
    )(hidden_states, Wg)

    choice_1d = choice.reshape(_B)

    grid_spec = pltpu.PrefetchScalarGridSpec(
        num_scalar_prefetch=1,
        grid=(_B, _F // _BF),
        in_specs=[
            pl.BlockSpec((1, _S, _D), lambda b, j, c: (b, 0, 0)),
            pl.BlockSpec((1, _D, _BF), lambda b, j, c: (c[b], 0, j)),
            pl.BlockSpec((1, _BF, _D), lambda b, j, c: (c[b], j, 0)),
        ],
        out_specs=pl.BlockSpec((1, _S, _D), lambda b, j, c: (b, 0, 0)),
        scratch_shapes=[pltpu.VMEM((_S, _D), jnp.bfloat16)],
    )
    out = pl.pallas_call(
        _moe_kernel,
        grid_spec=grid_spec,
        out_shape=jax.ShapeDtypeStruct((_B, _S, _D), jnp.float32),
        compiler_params=pltpu.CompilerParams(
            dimension_semantics=("arbitrary", "arbitrary"),
        ),
    )(choice_1d, hidden_states, W1, W2)

    return (out, logits)


# speedup vs baseline: 1.0603x; 1.0603x over previous
import jax
import jax.numpy as jnp
from jax.experimental import pallas as pl
from jax.experimental.pallas import tpu as pltpu

_B, _S, _D, _F, _E = 4, 2048, 1024, 4096, 8
_BF = 1024


def _gate_kernel(h_ref, wg_ref, logits_ref, choice_ref):
    hbar = jnp.mean(h_ref[...], axis=1)
    logits = jnp.dot(hbar, wg_ref[...], preferred_element_type=jnp.float32)
    logits_ref[...] = logits
    row_max = jnp.max(logits, axis=-1, keepdims=True)
    idx = jax.lax.broadcasted_iota(jnp.int32, logits.shape, 1)
    masked = jnp.where(logits == row_max, idx, _E)
    choice_ref[...] = jnp.min(masked, axis=-1, keepdims=True)


def _moe_kernel(choice_ref, h_ref, w1_ref, w2_ref, out_ref):
    fj = pl.program_id(1)

    @pl.when(fj == 0)
    def _init():
        out_ref[...] = jnp.zeros_like(out_ref)

    hmid = jax.nn.gelu(
        jnp.dot(h_ref[0], w1_ref[0], preferred_element_type=jnp.float32)
    )
    out_ref[0, :, :] += jnp.dot(
        hmid, w2_ref[0], preferred_element_type=jnp.float32
    )


def kernel(hidden_states, Wg, W1, W2):
    logits, choice = pl.pallas_call(
        _gate_kernel,
        out_shape=(
            jax.ShapeDtypeStruct((_B, _E), jnp.float32),
            jax.ShapeDtypeStruct((_B, 1), jnp.int32),
        ),
    )(hidden_states, Wg)

    choice_1d = choice.reshape(_B)

    grid_spec = pltpu.PrefetchScalarGridSpec(
        num_scalar_prefetch=1,
        grid=(_B, _F // _BF),
        in_specs=[
            pl.BlockSpec((1, _S, _D), lambda b, j, c: (b, 0, 0)),
            pl.BlockSpec((1, _D, _BF), lambda b, j, c: (c[b], 0, j)),
            pl.BlockSpec((1, _BF, _D), lambda b, j, c: (c[b], j, 0)),
        ],
        out_specs=pl.BlockSpec((1, _S, _D), lambda b, j, c: (b, 0, 0)),
    )
    out = pl.pallas_call(
        _moe_kernel,
        grid_spec=grid_spec,
        out_shape=jax.ShapeDtypeStruct((_B, _S, _D), jnp.float32),
        compiler_params=pltpu.CompilerParams(
            dimension_semantics=("arbitrary", "arbitrary"),
            vmem_limit_bytes=100 * 1024 * 1024,
        ),
    )(choice_1d, hidden_states, W1, W2)

    return (out, logits)
